# baseline (speedup 1.0000x reference)
import functools

import jax
import jax.numpy as jnp
from jax import lax
from jax.experimental import pallas as pl
from jax.experimental.pallas import tpu as pltpu
from jax.experimental.pallas import tpu_sc as plsc

N_EMBD = 64
BATCH = 16384

_info = plsc.get_sparse_core_info()
_NC, _NS, _NL = _info.num_cores, _info.num_subcores, _info.num_lanes
_NW = _NC * _NS
_B_PER_W = BATCH // _NW


@functools.partial(
    pl.kernel,
    mesh=plsc.VectorSubcoreMesh(core_axis_name="c", subcore_axis_name="s"),
    out_type=jax.ShapeDtypeStruct((BATCH, N_EMBD), jnp.float32),
    scratch_types=[
        pltpu.VMEM((_B_PER_W,), jnp.int32),
        pltpu.SemaphoreType.DMA,
    ],
)
def _gather_kernel(table_hbm, idx_hbm, out_hbm, idx_s, sem):
    wid = lax.axis_index("s") * _NC + lax.axis_index("c")
    base = wid * _B_PER_W
    pltpu.sync_copy(idx_hbm.at[wid], idx_s)

    def fire_group(g, _):
        v = idx_s[pl.ds(g * _NL, _NL)]
        for t in range(_NL):
            pltpu.async_copy(
                table_hbm.at[v[t]], out_hbm.at[base + g * _NL + t], sem
            )
        return 0

    lax.fori_loop(0, _B_PER_W // _NL, fire_group, 0)
    pltpu.make_async_copy(
        table_hbm.at[pl.ds(0, _B_PER_W)],
        out_hbm.at[pl.ds(base, _B_PER_W)],
        sem,
    ).wait()


def kernel(y, table):
    idx = y.astype(jnp.int32).reshape(_NW, _B_PER_W)
    return _gather_kernel(table, idx)


# speedup vs baseline: 1.6714x; 1.6714x over previous
import functools

import jax
import jax.numpy as jnp
from jax import lax
from jax.experimental import pallas as pl
from jax.experimental.pallas import tpu as pltpu
from jax.experimental.pallas import tpu_sc as plsc

N_EMBD = 64
BATCH = 16384

_info = plsc.get_sparse_core_info()
_NC, _NS, _NL = _info.num_cores, _info.num_subcores, _info.num_lanes
_NW = _NC * _NS
_B_PER_W = BATCH // _NW


@functools.partial(
    pl.kernel,
    mesh=plsc.VectorSubcoreMesh(core_axis_name="c", subcore_axis_name="s"),
    out_type=jax.ShapeDtypeStruct((BATCH, N_EMBD), jnp.float32),
    scratch_types=[
        pltpu.VMEM((_B_PER_W,), jnp.int32),
        pltpu.VMEM((_B_PER_W, N_EMBD), jnp.float32),
        pltpu.SemaphoreType.DMA,
    ],
)
def _gather_kernel(table_hbm, idx_hbm, out_hbm, idx_s, rows_v, sem):
    wid = lax.axis_index("s") * _NC + lax.axis_index("c")
    base = wid * _B_PER_W
    pltpu.sync_copy(idx_hbm.at[wid], idx_s)

    def fire_group(g, _):
        v = idx_s[pl.ds(g * _NL, _NL)]
        for t in range(_NL):
            pltpu.async_copy(
                table_hbm.at[v[t]], rows_v.at[g * _NL + t], sem
            )
        return 0

    lax.fori_loop(0, _B_PER_W // _NL, fire_group, 0)
    pltpu.make_async_copy(
        table_hbm.at[pl.ds(0, _B_PER_W)], rows_v, sem
    ).wait()
    pltpu.sync_copy(rows_v, out_hbm.at[pl.ds(base, _B_PER_W)])


def kernel(y, table):
    idx = y.astype(jnp.int32).reshape(_NW, _B_PER_W)
    return _gather_kernel(table, idx)
